# baseline (speedup 1.0000x reference)
import jax
import jax.numpy as jnp
from jax.experimental import pallas as pl

_FREQ_MASKS = 2
_TIME_MASKS = 10
_FREQ_WIDTH = 27
_TIME_WIDTH = 0.05


def _mask_mul_kernel(kt_ref, kf_ref, x_ref, o_ref):
    kt = kt_ref[0, 0, :]
    kf = kf_ref[0, 0, :]
    o_ref[0] = x_ref[0] * (kt[:, None] * kf[None, :])


def _keep_masks(length, B, T, F):
    key = jax.random.key(42)
    kf1, kf2, kt1, kt2 = jax.random.split(key, 4)
    x_left = jax.random.randint(kf1, (B, _FREQ_MASKS), 0, F - _FREQ_WIDTH + 1)
    wf = jax.random.randint(kf2, (B, _FREQ_MASKS), 0, _FREQ_WIDTH + 1)
    f_idx = jnp.arange(F)
    fmask = ((f_idx[None, None, :] >= x_left[:, :, None])
             & (f_idx[None, None, :] < (x_left + wf)[:, :, None])).any(axis=1)
    len32 = length.astype(jnp.int32)
    tw = jnp.maximum(1, (len32.astype(jnp.float32) * _TIME_WIDTH).astype(jnp.int32))
    y_max = jnp.maximum(1, len32 - tw)
    u1 = jax.random.uniform(kt1, (B, _TIME_MASKS))
    u2 = jax.random.uniform(kt2, (B, _TIME_MASKS))
    y_left = jnp.floor(u1 * (y_max[:, None] + 1).astype(jnp.float32)).astype(jnp.int32)
    y_left = jnp.minimum(y_left, y_max[:, None])
    wt = jnp.floor(u2 * (tw[:, None] + 1).astype(jnp.float32)).astype(jnp.int32)
    wt = jnp.minimum(wt, tw[:, None])
    t_idx = jnp.arange(T)
    tmask = ((t_idx[None, None, :] >= y_left[:, :, None])
             & (t_idx[None, None, :] < (y_left + wt)[:, :, None])).any(axis=1)
    keep_t = jnp.logical_not(tmask).astype(jnp.float32)
    keep_f = jnp.logical_not(fmask).astype(jnp.float32)
    return keep_t, keep_f


def kernel(input_spec, length):
    B, T, F = input_spec.shape
    keep_t, keep_f = _keep_masks(length, B, T, F)

    T_BLK = 1024
    grid = (B, T // T_BLK)
    out = pl.pallas_call(
        _mask_mul_kernel,
        grid=grid,
        in_specs=[
            pl.BlockSpec((1, 1, T_BLK), lambda i, j: (i, 0, j)),
            pl.BlockSpec((1, 1, F), lambda i, j: (i, 0, 0)),
            pl.BlockSpec((1, T_BLK, F), lambda i, j: (i, j, 0)),
        ],
        out_specs=pl.BlockSpec((1, T_BLK, F), lambda i, j: (i, j, 0)),
        out_shape=jax.ShapeDtypeStruct((B, T, F), input_spec.dtype),
    )(keep_t[:, None, :], keep_f[:, None, :], input_spec)
    return (out, length)


# speedup vs baseline: 1.6960x; 1.6960x over previous
import jax
import jax.numpy as jnp
from jax.experimental import pallas as pl

_FREQ_MASKS = 2
_TIME_MASKS = 10
_FREQ_WIDTH = 27
_TIME_WIDTH = 0.05


def _mask_mul_kernel(kt_ref, kf_ref, x_ref, o_ref):
    kt = kt_ref[0, 0, :]
    kf = kf_ref[0, 0, :]
    o_ref[0] = x_ref[0] * (kt[:, None] * kf[None, :])


def _keep_masks(length, B, T, F):
    key = jax.random.key(42)
    kf1, kf2, kt1, kt2 = jax.random.split(key, 4)
    x_left = jax.random.randint(kf1, (B, _FREQ_MASKS), 0, F - _FREQ_WIDTH + 1)
    wf = jax.random.randint(kf2, (B, _FREQ_MASKS), 0, _FREQ_WIDTH + 1)
    f_idx = jnp.arange(F)
    fmask = ((f_idx[None, None, :] >= x_left[:, :, None])
             & (f_idx[None, None, :] < (x_left + wf)[:, :, None])).any(axis=1)
    len32 = length.astype(jnp.int32)
    tw = jnp.maximum(1, (len32.astype(jnp.float32) * _TIME_WIDTH).astype(jnp.int32))
    y_max = jnp.maximum(1, len32 - tw)
    u1 = jax.random.uniform(kt1, (B, _TIME_MASKS))
    u2 = jax.random.uniform(kt2, (B, _TIME_MASKS))
    y_left = jnp.floor(u1 * (y_max[:, None] + 1).astype(jnp.float32)).astype(jnp.int32)
    y_left = jnp.minimum(y_left, y_max[:, None])
    wt = jnp.floor(u2 * (tw[:, None] + 1).astype(jnp.float32)).astype(jnp.int32)
    wt = jnp.minimum(wt, tw[:, None])
    t_idx = jnp.arange(T)
    tmask = ((t_idx[None, None, :] >= y_left[:, :, None])
             & (t_idx[None, None, :] < (y_left + wt)[:, :, None])).any(axis=1)
    keep_t = jnp.logical_not(tmask).astype(jnp.float32)
    keep_f = jnp.logical_not(fmask).astype(jnp.float32)
    return keep_t, keep_f


def kernel(input_spec, length):
    B, T, F = input_spec.shape
    keep_t, keep_f = _keep_masks(length, B, T, F)

    T_BLK = 4096
    grid = (B, T // T_BLK)
    out = pl.pallas_call(
        _mask_mul_kernel,
        grid=grid,
        in_specs=[
            pl.BlockSpec((1, 1, T_BLK), lambda i, j: (i, 0, j)),
            pl.BlockSpec((1, 1, F), lambda i, j: (i, 0, 0)),
            pl.BlockSpec((1, T_BLK, F), lambda i, j: (i, j, 0)),
        ],
        out_specs=pl.BlockSpec((1, T_BLK, F), lambda i, j: (i, j, 0)),
        out_shape=jax.ShapeDtypeStruct((B, T, F), input_spec.dtype),
    )(keep_t[:, None, :], keep_f[:, None, :], input_spec)
    return (out, length)


# speedup vs baseline: 2.2332x; 1.3167x over previous
import numpy as np
import jax
import jax.numpy as jnp
from jax.experimental import pallas as pl
from jax.experimental.pallas import tpu as pltpu

_FREQ_MASKS = 2
_TIME_MASKS = 10
_FREQ_WIDTH = 27
_TIME_WIDTH = 0.05


def _trace_time_constants(B, F):
    with jax.ensure_compile_time_eval():
        key = jax.random.key(42)
        kf1, kf2, kt1, kt2 = jax.random.split(key, 4)
        x_left = jax.random.randint(kf1, (B, _FREQ_MASKS), 0, F - _FREQ_WIDTH + 1)
        wf = jax.random.randint(kf2, (B, _FREQ_MASKS), 0, _FREQ_WIDTH + 1)
        f_idx = jnp.arange(F)
        fmask = ((f_idx[None, None, :] >= x_left[:, :, None])
                 & (f_idx[None, None, :] < (x_left + wf)[:, :, None])).any(axis=1)
        keep_f = jnp.logical_not(fmask).astype(jnp.float32)
        u1 = jax.random.uniform(kt1, (B, _TIME_MASKS))
        u2 = jax.random.uniform(kt2, (B, _TIME_MASKS))
    return (np.asarray(keep_f).reshape(B, 1, F),
            np.asarray(u1), np.asarray(u2))


def _mask_mul_kernel(len_ref, u1_ref, u2_ref, kf_ref, x_ref, o_ref, *, t_blk):
    b = pl.program_id(0)
    j = pl.program_id(1)
    lenb = len_ref[b]
    len_f = lenb.astype(jnp.float32)
    tw = jnp.maximum(1, (len_f * _TIME_WIDTH).astype(jnp.int32))
    y_max = jnp.maximum(1, lenb - tw)
    ymf = (y_max + 1).astype(jnp.float32)
    twf = (tw + 1).astype(jnp.float32)
    ti = jax.lax.broadcasted_iota(jnp.int32, (1, t_blk), 1) + j * t_blk
    masked = ti < 0
    for m in range(_TIME_MASKS):
        u1 = u1_ref[b, m]
        u2 = u2_ref[b, m]
        y = jnp.minimum(jnp.floor(u1 * ymf).astype(jnp.int32), y_max)
        w = jnp.minimum(jnp.floor(u2 * twf).astype(jnp.int32), tw)
        masked = masked | ((ti >= y) & (ti < y + w))
    kt = jnp.where(masked, 0.0, 1.0)
    kf = kf_ref[0]
    o_ref[0] = x_ref[0] * (kt.reshape(t_blk, 1) * kf)


def kernel(input_spec, length):
    B, T, F = input_spec.shape
    keep_f, u1, u2 = _trace_time_constants(B, F)
    len32 = length.astype(jnp.int32)

    T_BLK = 4096
    grid = (B, T // T_BLK)
    import functools
    out = pl.pallas_call(
        functools.partial(_mask_mul_kernel, t_blk=T_BLK),
        grid_spec=pltpu.PrefetchScalarGridSpec(
            num_scalar_prefetch=3,
            grid=grid,
            in_specs=[
                pl.BlockSpec((1, 1, F), lambda i, j, *_: (i, 0, 0)),
                pl.BlockSpec((1, T_BLK, F), lambda i, j, *_: (i, j, 0)),
            ],
            out_specs=pl.BlockSpec((1, T_BLK, F), lambda i, j, *_: (i, j, 0)),
        ),
        out_shape=jax.ShapeDtypeStruct((B, T, F), input_spec.dtype),
    )(len32, jnp.asarray(u1), jnp.asarray(u2), jnp.asarray(keep_f), input_spec)
    return (out, length)


# speedup vs baseline: 2.3955x; 1.0727x over previous
import functools

import numpy as np
import jax
import jax.numpy as jnp
from jax.experimental import pallas as pl
from jax.experimental.pallas import tpu as pltpu

_FREQ_MASKS = 2
_TIME_MASKS = 10
_FREQ_WIDTH = 27
_TIME_WIDTH = 0.05
_WIN = 256


def _trace_time_constants(B, F):
    with jax.ensure_compile_time_eval():
        key = jax.random.key(42)
        kf1, kf2, kt1, kt2 = jax.random.split(key, 4)
        x_left = jax.random.randint(kf1, (B, _FREQ_MASKS), 0, F - _FREQ_WIDTH + 1)
        wf = jax.random.randint(kf2, (B, _FREQ_MASKS), 0, _FREQ_WIDTH + 1)
        f_idx = jnp.arange(F)
        fmask = ((f_idx[None, None, :] >= x_left[:, :, None])
                 & (f_idx[None, None, :] < (x_left + wf)[:, :, None])).any(axis=1)
        keep_f = jnp.logical_not(fmask).astype(jnp.float32)
        u1 = jax.random.uniform(kt1, (B, _TIME_MASKS))
        u2 = jax.random.uniform(kt2, (B, _TIME_MASKS))
    return (np.asarray(keep_f).reshape(B, 1, F),
            np.asarray(u1), np.asarray(u2))


def _mask_kernel(len_ref, u1_ref, u2_ref, kf_ref, x_ref, o_ref, *, t_blk):
    b = pl.program_id(0)
    o_ref[0] = x_ref[0] * kf_ref[0]
    lenb = len_ref[b]
    len_f = lenb.astype(jnp.float32)
    tw = jnp.maximum(1, (len_f * _TIME_WIDTH).astype(jnp.int32))
    y_max = jnp.maximum(1, lenb - tw)
    ymf = (y_max + 1).astype(jnp.float32)
    twf = (tw + 1).astype(jnp.float32)
    for m in range(_TIME_MASKS):
        u1 = u1_ref[b, m]
        u2 = u2_ref[b, m]
        y = jnp.minimum(jnp.floor(u1 * ymf).astype(jnp.int32), y_max)
        w = jnp.minimum(jnp.floor(u2 * twf).astype(jnp.int32), tw)
        s = jnp.minimum((y // 8) * 8, t_blk - _WIN)
        ti = jax.lax.broadcasted_iota(jnp.int32, (_WIN, 1), 0) + s
        keepm = jnp.where((ti >= y) & (ti < y + w), 0.0, 1.0)
        o_ref[0, pl.ds(s, _WIN), :] = o_ref[0, pl.ds(s, _WIN), :] * keepm


def kernel(input_spec, length):
    B, T, F = input_spec.shape
    keep_f, u1, u2 = _trace_time_constants(B, F)
    len32 = length.astype(jnp.int32)

    T_BLK = T
    grid = (B,)
    out = pl.pallas_call(
        functools.partial(_mask_kernel, t_blk=T_BLK),
        grid_spec=pltpu.PrefetchScalarGridSpec(
            num_scalar_prefetch=3,
            grid=grid,
            in_specs=[
                pl.BlockSpec((1, 1, F), lambda i, *_: (i, 0, 0)),
                pl.BlockSpec((1, T_BLK, F), lambda i, *_: (i, 0, 0)),
            ],
            out_specs=pl.BlockSpec((1, T_BLK, F), lambda i, *_: (i, 0, 0)),
        ),
        out_shape=jax.ShapeDtypeStruct((B, T, F), input_spec.dtype),
    )(len32, jnp.asarray(u1), jnp.asarray(u2), jnp.asarray(keep_f), input_spec)
    return (out, length)


# speedup vs baseline: 2.6946x; 1.1249x over previous
import functools

import numpy as np
import jax
import jax.numpy as jnp
from jax.experimental import pallas as pl
from jax.experimental.pallas import tpu as pltpu

_FREQ_MASKS = 2
_TIME_MASKS = 10
_FREQ_WIDTH = 27
_TIME_WIDTH = 0.05
_WIN = 256


def _trace_time_constants(B, F):
    with jax.ensure_compile_time_eval():
        key = jax.random.key(42)
        kf1, kf2, kt1, kt2 = jax.random.split(key, 4)
        x_left = jax.random.randint(kf1, (B, _FREQ_MASKS), 0, F - _FREQ_WIDTH + 1)
        wf = jax.random.randint(kf2, (B, _FREQ_MASKS), 0, _FREQ_WIDTH + 1)
        f_idx = jnp.arange(F)
        fmask = ((f_idx[None, None, :] >= x_left[:, :, None])
                 & (f_idx[None, None, :] < (x_left + wf)[:, :, None])).any(axis=1)
        keep_f = jnp.logical_not(fmask).astype(jnp.float32)
        u1 = jax.random.uniform(kt1, (B, _TIME_MASKS))
        u2 = jax.random.uniform(kt2, (B, _TIME_MASKS))
    return (np.asarray(keep_f).reshape(B, 1, F),
            np.asarray(u1), np.asarray(u2))


def _mask_kernel(len_ref, u1_ref, u2_ref, kf_ref, x_ref, o_ref, *, t_blk, bb):
    i = pl.program_id(0)
    o_ref[...] = x_ref[...] * kf_ref[...]
    for k in range(bb):
        b = i * bb + k
        lenb = len_ref[b]
        len_f = lenb.astype(jnp.float32)
        tw = jnp.maximum(1, (len_f * _TIME_WIDTH).astype(jnp.int32))
        y_max = jnp.maximum(1, lenb - tw)
        ymf = (y_max + 1).astype(jnp.float32)
        twf = (tw + 1).astype(jnp.float32)
        for m in range(_TIME_MASKS):
            u1 = u1_ref[b, m]
            u2 = u2_ref[b, m]
            y = jnp.minimum(jnp.floor(u1 * ymf).astype(jnp.int32), y_max)
            w = jnp.minimum(jnp.floor(u2 * twf).astype(jnp.int32), tw)
            s = jnp.minimum((y // 8) * 8, t_blk - _WIN)
            ti = jax.lax.broadcasted_iota(jnp.int32, (_WIN, 1), 0) + s
            keepm = jnp.where((ti >= y) & (ti < y + w), 0.0, 1.0)
            o_ref[k, pl.ds(s, _WIN), :] = o_ref[k, pl.ds(s, _WIN), :] * keepm


def kernel(input_spec, length):
    B, T, F = input_spec.shape
    keep_f, u1, u2 = _trace_time_constants(B, F)
    len32 = length.astype(jnp.int32)

    T_BLK = T
    BB = 2
    grid = (B // BB,)
    out = pl.pallas_call(
        functools.partial(_mask_kernel, t_blk=T_BLK, bb=BB),
        grid_spec=pltpu.PrefetchScalarGridSpec(
            num_scalar_prefetch=3,
            grid=grid,
            in_specs=[
                pl.BlockSpec((BB, 1, F), lambda i, *_: (i, 0, 0)),
                pl.BlockSpec((BB, T_BLK, F), lambda i, *_: (i, 0, 0)),
            ],
            out_specs=pl.BlockSpec((BB, T_BLK, F), lambda i, *_: (i, 0, 0)),
        ),
        out_shape=jax.ShapeDtypeStruct((B, T, F), input_spec.dtype),
    )(len32, jnp.asarray(u1), jnp.asarray(u2), jnp.asarray(keep_f), input_spec)
    return (out, length)


# speedup vs baseline: 2.7285x; 1.0126x over previous
import functools

import numpy as np
import jax
import jax.numpy as jnp
from jax.experimental import pallas as pl
from jax.experimental.pallas import tpu as pltpu

_FREQ_MASKS = 2
_TIME_MASKS = 10
_FREQ_WIDTH = 27
_TIME_WIDTH = 0.05
_WIN = 256


def _trace_time_constants(B, F):
    with jax.ensure_compile_time_eval():
        key = jax.random.key(42)
        kf1, kf2, kt1, kt2 = jax.random.split(key, 4)
        x_left = jax.random.randint(kf1, (B, _FREQ_MASKS), 0, F - _FREQ_WIDTH + 1)
        wf = jax.random.randint(kf2, (B, _FREQ_MASKS), 0, _FREQ_WIDTH + 1)
        f_idx = jnp.arange(F)
        fmask = ((f_idx[None, None, :] >= x_left[:, :, None])
                 & (f_idx[None, None, :] < (x_left + wf)[:, :, None])).any(axis=1)
        keep_f = jnp.logical_not(fmask).astype(jnp.float32)
        u1 = jax.random.uniform(kt1, (B, _TIME_MASKS))
        u2 = jax.random.uniform(kt2, (B, _TIME_MASKS))
    return (np.asarray(keep_f).reshape(B, 1, F),
            np.asarray(u1), np.asarray(u2))


def _mask_kernel(len_ref, u1_ref, u2_ref, kf_ref, x_ref, o_ref, *, t_blk, bb):
    i = pl.program_id(0)
    o_ref[...] = x_ref[...] * kf_ref[...]
    for k in range(bb):
        b = i * bb + k
        lenb = len_ref[b]
        len_f = lenb.astype(jnp.float32)
        tw = jnp.maximum(1, (len_f * _TIME_WIDTH).astype(jnp.int32))
        y_max = jnp.maximum(1, lenb - tw)
        ymf = (y_max + 1).astype(jnp.float32)
        twf = (tw + 1).astype(jnp.float32)
        for m in range(_TIME_MASKS):
            u1 = u1_ref[b, m]
            u2 = u2_ref[b, m]
            y = jnp.minimum(jnp.floor(u1 * ymf).astype(jnp.int32), y_max)
            w = jnp.minimum(jnp.floor(u2 * twf).astype(jnp.int32), tw)
            s = jnp.minimum((y // 8) * 8, t_blk - _WIN)
            ti = jax.lax.broadcasted_iota(jnp.int32, (_WIN, 1), 0) + s
            keepm = jnp.where((ti >= y) & (ti < y + w), 0.0, 1.0)
            o_ref[k, pl.ds(s, _WIN), :] = o_ref[k, pl.ds(s, _WIN), :] * keepm


def kernel(input_spec, length):
    B, T, F = input_spec.shape
    keep_f, u1, u2 = _trace_time_constants(B, F)
    len32 = length.astype(jnp.int32)

    T_BLK = T
    BB = 4
    grid = (B // BB,)
    out = pl.pallas_call(
        functools.partial(_mask_kernel, t_blk=T_BLK, bb=BB),
        grid_spec=pltpu.PrefetchScalarGridSpec(
            num_scalar_prefetch=3,
            grid=grid,
            in_specs=[
                pl.BlockSpec((BB, 1, F), lambda i, *_: (i, 0, 0)),
                pl.BlockSpec((BB, T_BLK, F), lambda i, *_: (i, 0, 0)),
            ],
            out_specs=pl.BlockSpec((BB, T_BLK, F), lambda i, *_: (i, 0, 0)),
        ),
        out_shape=jax.ShapeDtypeStruct((B, T, F), input_spec.dtype),
    )(len32, jnp.asarray(u1), jnp.asarray(u2), jnp.asarray(keep_f), input_spec)
    return (out, length)
